# Initial kernel scaffold; baseline (speedup 1.0000x reference)
import jax
import jax.numpy as jnp
from jax.experimental import pallas as pl


def kernel(x, t, W_static, W_dyn0, W_dyn1, W_dyn2):
    raise NotImplementedError("write your pallas kernel here")


# speedup vs baseline: 11.0249x; 11.0249x over previous
import functools

import numpy as np
import jax
import jax.numpy as jnp
from jax import lax
from jax.experimental import pallas as pl
from jax.experimental.pallas import tpu as pltpu
from jax.experimental.pallas import tpu_sc as plsc

N_LEVELS = 8
F = 4
BASE_RES = 512.0
MAX_RES = 32768.0
PER_LEVEL_SCALE = float(np.exp2(np.log2(MAX_RES / BASE_RES) / (N_LEVELS - 1)))
TIME_RES = 8
NUM_BASIS = 4
N_PTS = 131072

PRIME1 = int(np.uint32(2654435761).astype(np.int32))
PRIME2 = int(np.uint32(805459861).astype(np.int32))

SCALES = [
    float(np.exp2(l * np.log2(PER_LEVEL_SCALE)) * BASE_RES - 1.0)
    for l in range(N_LEVELS)
]

T_STATIC = 2 ** 19
MASK_S = T_STATIC - 1

NW = 32
PPT = N_PTS // NW
CH = 512
CH8 = CH * 8


def _reduce_body(c_ref, a_ref, b_ref, o_ref):
    a = a_ref[...]
    b = b_ref[...]
    acc = (
        c_ref[0] * a[0:1, :] + c_ref[1] * a[1:2, :]
        + c_ref[2] * a[2:3, :] + c_ref[3] * a[3:4, :]
        + c_ref[4] * b[0:1, :] + c_ref[5] * b[1:2, :]
        + c_ref[6] * b[2:3, :] + c_ref[7] * b[3:4, :]
    )
    o_ref[...] = acc


def _reduce_pair(cvec, a4, b4):
    m = a4.shape[1]
    blk = 8192
    return pl.pallas_call(
        _reduce_body,
        grid=(m // blk,),
        in_specs=[
            pl.BlockSpec(memory_space=pltpu.SMEM),
            pl.BlockSpec((4, blk), lambda i: (0, i)),
            pl.BlockSpec((4, blk), lambda i: (0, i)),
        ],
        out_specs=pl.BlockSpec((1, blk), lambda i: (0, i)),
        out_shape=jax.ShapeDtypeStruct((1, m), jnp.float32),
    )(cvec, a4, b4)


def _sc_body(xT, scal, wst, r0, r1, r2, s_out, d_out,
             x0b, x1b, x2b, sclb, ridxb, idxb, wb, gbuf, stg, rbuf, colb, sem):
    wid = lax.axis_index("s") * 2 + lax.axis_index("c")
    base = wid * PPT

    pltpu.sync_copy(xT.at[pl.ds(base, PPT)], x0b)
    pltpu.sync_copy(xT.at[pl.ds(N_PTS + base, PPT)], x1b)
    pltpu.sync_copy(xT.at[pl.ds(2 * N_PTS + base, PPT)], x2b)
    pltpu.sync_copy(scal, sclb)

    iota = lax.iota(jnp.int32, 16)
    q4 = lax.shift_right_logical(iota, 2)
    f4 = lax.bitwise_and(iota, 3)

    for l in range(N_LEVELS):
        scale = SCALES[l]
        lvl_off = l * T_STATIC

        def chunk_body(ch, _, scale=scale, lvl_off=lvl_off, sl=l):
            pbase = ch * CH

            def hash_body(i, _):
                n0 = pbase + i * 16
                o = i * 16
                p0 = x0b[pl.ds(n0, 16)] * scale + 0.5
                p1 = x1b[pl.ds(n0, 16)] * scale + 0.5
                p2 = x2b[pl.ds(n0, 16)] * scale + 0.5
                i0 = p0.astype(jnp.int32)
                i1_ = p1.astype(jnp.int32)
                i2_ = p2.astype(jnp.int32)
                f0 = p0 - i0.astype(jnp.float32)
                f1 = p1 - i1_.astype(jnp.float32)
                f2 = p2 - i2_.astype(jnp.float32)
                g0 = 1.0 - f0
                g1 = 1.0 - f1
                g2 = 1.0 - f2
                h1a = i1_ * PRIME1
                h1b = h1a + PRIME1
                h2a = i2_ * PRIME2
                h2b = h2a + PRIME2
                i0b = i0 + 1
                for c in range(8):
                    hx = i0b if (c & 1) else i0
                    hy = h1b if (c & 2) else h1a
                    hz = h2b if (c & 4) else h2a
                    idx = lax.bitwise_and(
                        lax.bitwise_xor(lax.bitwise_xor(hx, hy), hz), MASK_S
                    ) + lvl_off
                    wx = f0 if (c & 1) else g0
                    wy = f1 if (c & 2) else g1
                    wz = f2 if (c & 4) else g2
                    ridxb[pl.ds(c * CH + o, 16)] = idx
                    wb[pl.ds(c * CH + o, 16)] = wx * wy * wz
                return _

            lax.fori_loop(0, CH // 16, hash_body, None)

            def expand_body(g, _):
                for q in range(4):
                    piece = plsc.load_gather(ridxb, [g * 16 + q * 4 + q4])
                    idxb[pl.ds(g * 64 + q * 16, 16)] = piece * 4 + f4
                return _

            lax.fori_loop(0, CH8 // 16, expand_body, None)

            pltpu.async_copy(wst.at[idxb], gbuf, sem).wait()

            def acc_body(g, _):
                gb16 = g * 16
                for q in range(4):
                    off = gb16 + q * 4
                    acc = jnp.zeros((16,), jnp.float32)
                    for c in range(8):
                        row = gbuf[pl.ds((c * CH + off) * 4, 16)]
                        wexp = plsc.load_gather(wb, [c * CH + off + q4])
                        acc = acc + wexp * row
                    stg[pl.ds(gb16 * 4 + q * 16, 16)] = acc
                return _

            lax.fori_loop(0, CH // 16, acc_body, None)
            pltpu.sync_copy(
                stg,
                s_out.at[pl.ds(sl * 4 * N_PTS + (base + pbase) * 4, CH * 4)],
            )
            return _

        lax.fori_loop(0, PPT // CH, chunk_body, None)

    planes = (
        (r0, 32768, x0b, x1b),
        (r1, 8192, x0b, x2b),
        (r2, 8192, x1b, x2b),
    )
    for p, (rref, tsz, xab, xbb) in enumerate(planes):
        maskp = tsz - 1

        def lvl_body(l, _, rref=rref, tsz=tsz, xab=xab, xbb=xbb,
                     maskp=maskp, p=p):
            pltpu.sync_copy(rref.at[pl.ds(l * tsz, tsz)], rbuf.at[pl.ds(0, tsz)])
            scl = plsc.load_gather(sclb, [jnp.full((16,), l, jnp.int32)])

            def pt_body(g, _):
                n0 = g * 16
                pa = xab[pl.ds(n0, 16)] * scl + 0.5
                pb = xbb[pl.ds(n0, 16)] * scl + 0.5
                ia = pa.astype(jnp.int32)
                ib = pb.astype(jnp.int32)
                fa = pa - ia.astype(jnp.float32)
                fb = pb - ib.astype(jnp.float32)
                ga = 1.0 - fa
                gb2 = 1.0 - fb
                hb0 = ib * PRIME1
                hb1 = hb0 + PRIME1
                ia1 = ia + 1
                i00 = lax.bitwise_and(lax.bitwise_xor(ia, hb0), maskp)
                i10 = lax.bitwise_and(lax.bitwise_xor(ia1, hb0), maskp)
                i01 = lax.bitwise_and(lax.bitwise_xor(ia, hb1), maskp)
                i11 = lax.bitwise_and(lax.bitwise_xor(ia1, hb1), maskp)
                v = (ga * gb2) * plsc.load_gather(rbuf, [i00])
                v = v + (fa * gb2) * plsc.load_gather(rbuf, [i10])
                v = v + (ga * fb) * plsc.load_gather(rbuf, [i01])
                v = v + (fa * fb) * plsc.load_gather(rbuf, [i11])
                colb[pl.ds(n0, 16)] = v
                return _

            lax.fori_loop(0, PPT // 16, pt_body, None)
            pltpu.sync_copy(
                colb, d_out.at[pl.ds((p * 8 + l) * N_PTS + base, PPT)]
            )
            return _

        lax.fori_loop(0, N_LEVELS, lvl_body, None)


def _sc_call(xT, scal, wst2, R0, R1, R2):
    mesh = plsc.VectorSubcoreMesh(
        core_axis_name="c", subcore_axis_name="s", num_cores=2, num_subcores=16
    )
    fn = pl.kernel(
        _sc_body,
        out_type=(
            jax.ShapeDtypeStruct((N_LEVELS * 4 * N_PTS,), jnp.float32),
            jax.ShapeDtypeStruct((24 * N_PTS,), jnp.float32),
        ),
        mesh=mesh,
        compiler_params=pltpu.CompilerParams(
            needs_layout_passes=False, use_tc_tiling_on_sc=False
        ),
        scratch_types=[
            pltpu.VMEM((PPT,), jnp.float32),
            pltpu.VMEM((PPT,), jnp.float32),
            pltpu.VMEM((PPT,), jnp.float32),
            pltpu.VMEM((16,), jnp.float32),
            pltpu.VMEM((CH8,), jnp.int32),
            pltpu.VMEM((CH8 * 4,), jnp.int32),
            pltpu.VMEM((CH8,), jnp.float32),
            pltpu.VMEM((CH8 * 4,), jnp.float32),
            pltpu.VMEM((CH * 4,), jnp.float32),
            pltpu.VMEM((32768,), jnp.float32),
            pltpu.VMEM((PPT,), jnp.float32),
            pltpu.SemaphoreType.DMA,
        ],
    )
    return fn(xT, scal, wst2, R0, R1, R2)


def kernel(x, t, W_static, W_dyn0, W_dyn1, W_dyn2):
    ts = t[0]
    s = ts * (TIME_RES - 1.0)
    resid = (ts * float(TIME_RES) - s) - ts
    fl_s = jnp.floor(s)
    on_grid = s == fl_s
    i1 = fl_s.astype(jnp.int32) - jnp.where(on_grid & (resid < 0), 1, 0)
    i2 = jnp.ceil(s).astype(jnp.int32) + jnp.where(on_grid & (resid > 0), 1, 0)
    same = i1 == i2
    w1 = jnp.where(same, jnp.float32(1.0), i2.astype(jnp.float32) - s)
    w2 = s - i1.astype(jnp.float32)

    tn = [i / (NUM_BASIS - 1) for i in range(NUM_BASIS)]
    bs = []
    for j in range(NUM_BASIS):
        b = jnp.float32(1.0)
        for m in range(NUM_BASIS):
            if m != j:
                b = b * (ts - tn[m]) / (tn[j] - tn[m])
        bs.append(b)
    cvec = jnp.stack([w1 * bj for bj in bs] + [w2 * bj for bj in bs])
    cvec = cvec.astype(jnp.float32)

    def slice_t(w, i):
        return lax.dynamic_index_in_dim(w, i, axis=0, keepdims=False)

    rs = []
    for wdyn in (W_dyn0, W_dyn1, W_dyn2):
        tsz = wdyn.shape[2]
        a4 = slice_t(wdyn, i1).reshape(N_LEVELS * tsz, F).T
        b4 = slice_t(wdyn, i2).reshape(N_LEVELS * tsz, F).T
        rs.append(_reduce_pair(cvec, a4, b4).reshape(N_LEVELS * tsz))

    xT = x.T.reshape(3 * N_PTS)
    scal = jnp.asarray(SCALES + [0.0] * 8, dtype=jnp.float32)
    wst2 = W_static.reshape(N_LEVELS * T_STATIC * F)

    s_flat, d_mat = _sc_call(xT, scal, wst2, rs[0], rs[1], rs[2])

    feat_static = (
        s_flat.reshape(N_LEVELS, N_PTS, F).transpose(1, 0, 2).reshape(N_PTS, 32)
    )
    feat_dynamic = d_mat.reshape(24, N_PTS).T
    return (feat_static, feat_dynamic)


# speedup vs baseline: 34.1133x; 3.0942x over previous
import functools

import numpy as np
import jax
import jax.numpy as jnp
from jax import lax
from jax.experimental import pallas as pl
from jax.experimental.pallas import tpu as pltpu
from jax.experimental.pallas import tpu_sc as plsc

N_LEVELS = 8
F = 4
BASE_RES = 512.0
MAX_RES = 32768.0
PER_LEVEL_SCALE = float(np.exp2(np.log2(MAX_RES / BASE_RES) / (N_LEVELS - 1)))
TIME_RES = 8
NUM_BASIS = 4
N_PTS = 131072

PRIME1 = int(np.uint32(2654435761).astype(np.int32))
PRIME2 = int(np.uint32(805459861).astype(np.int32))

SCALES = [
    float(np.exp2(l * np.log2(PER_LEVEL_SCALE)) * BASE_RES - 1.0)
    for l in range(N_LEVELS)
]

T_STATIC = 2 ** 19
MASK_S = T_STATIC - 1

NW = 32
PPT = N_PTS // NW
CH = 512
CH8 = CH * 8


def _reduce_body(c_ref, a_ref, b_ref, o_ref):
    a = a_ref[...]
    b = b_ref[...]
    acc = (
        c_ref[0] * a[:, 0, :] + c_ref[1] * a[:, 1, :]
        + c_ref[2] * a[:, 2, :] + c_ref[3] * a[:, 3, :]
        + c_ref[4] * b[:, 0, :] + c_ref[5] * b[:, 1, :]
        + c_ref[6] * b[:, 2, :] + c_ref[7] * b[:, 3, :]
    )
    o_ref[...] = acc


def _reduce_pair(cvec, a3, b3):
    t = a3.shape[2]
    blk = 4096
    return pl.pallas_call(
        _reduce_body,
        grid=(t // blk,),
        in_specs=[
            pl.BlockSpec(memory_space=pltpu.SMEM),
            pl.BlockSpec((N_LEVELS, 4, blk), lambda i: (0, 0, i)),
            pl.BlockSpec((N_LEVELS, 4, blk), lambda i: (0, 0, i)),
        ],
        out_specs=pl.BlockSpec((N_LEVELS, blk), lambda i: (0, i)),
        out_shape=jax.ShapeDtypeStruct((N_LEVELS, t), jnp.float32),
    )(cvec, a3, b3)


def _sc_body(xT, scal, wst, r0, r1, r2, s_out, d_out,
             x0b, x1b, x2b, sclb, ridxb, idxb, wb, gbuf, stg, rbuf, colb, sem):
    wid = lax.axis_index("s") * 2 + lax.axis_index("c")
    base = wid * PPT

    pltpu.sync_copy(xT.at[pl.ds(base, PPT)], x0b)
    pltpu.sync_copy(xT.at[pl.ds(N_PTS + base, PPT)], x1b)
    pltpu.sync_copy(xT.at[pl.ds(2 * N_PTS + base, PPT)], x2b)
    pltpu.sync_copy(scal, sclb)

    iota = lax.iota(jnp.int32, 16)
    q4 = lax.shift_right_logical(iota, 2)
    f4 = lax.bitwise_and(iota, 3)

    for l in range(N_LEVELS):
        scale = SCALES[l]
        lvl_off = l * 4 * T_STATIC

        def chunk_body(ch, _, scale=scale, lvl_off=lvl_off, sl=l):
            pbase = ch * CH

            def hash_body(i, _):
                n0 = pbase + i * 16
                o = i * 16
                p0 = x0b[pl.ds(n0, 16)] * scale + 0.5
                p1 = x1b[pl.ds(n0, 16)] * scale + 0.5
                p2 = x2b[pl.ds(n0, 16)] * scale + 0.5
                i0 = p0.astype(jnp.int32)
                i1_ = p1.astype(jnp.int32)
                i2_ = p2.astype(jnp.int32)
                f0 = p0 - i0.astype(jnp.float32)
                f1 = p1 - i1_.astype(jnp.float32)
                f2 = p2 - i2_.astype(jnp.float32)
                g0 = 1.0 - f0
                g1 = 1.0 - f1
                g2 = 1.0 - f2
                h1a = i1_ * PRIME1
                h1b = h1a + PRIME1
                h2a = i2_ * PRIME2
                h2b = h2a + PRIME2
                i0b = i0 + 1
                for c in range(8):
                    hx = i0b if (c & 1) else i0
                    hy = h1b if (c & 2) else h1a
                    hz = h2b if (c & 4) else h2a
                    idx = lax.bitwise_and(
                        lax.bitwise_xor(lax.bitwise_xor(hx, hy), hz), MASK_S
                    ) + lvl_off
                    wx = f0 if (c & 1) else g0
                    wy = f1 if (c & 2) else g1
                    wz = f2 if (c & 4) else g2
                    ridxb[pl.ds(c * CH + o, 16)] = idx
                    wb[pl.ds(c * CH + o, 16)] = wx * wy * wz
                return _

            lax.fori_loop(0, CH // 16, hash_body, None)

            def expand_body(g, _):
                for q in range(4):
                    piece = plsc.load_gather(ridxb, [g * 16 + q * 4 + q4])
                    idxb[pl.ds(g * 64 + q * 16, 16)] = piece + f4 * T_STATIC
                return _

            lax.fori_loop(0, CH8 // 16, expand_body, None)

            pltpu.async_copy(wst.at[idxb], gbuf, sem).wait()

            def acc_body(g, _):
                gb16 = g * 16
                for q in range(4):
                    off = gb16 + q * 4
                    acc = jnp.zeros((16,), jnp.float32)
                    for c in range(8):
                        row = gbuf[pl.ds((c * CH + off) * 4, 16)]
                        wexp = plsc.load_gather(wb, [c * CH + off + q4])
                        acc = acc + wexp * row
                    stg[pl.ds(gb16 * 4 + q * 16, 16)] = acc
                return _

            lax.fori_loop(0, CH // 16, acc_body, None)
            pltpu.sync_copy(
                stg,
                s_out.at[pl.ds(sl * 4 * N_PTS + (base + pbase) * 4, CH * 4)],
            )
            return _

        lax.fori_loop(0, PPT // CH, chunk_body, None)

    planes = (
        (r0, 32768, x0b, x1b),
        (r1, 8192, x0b, x2b),
        (r2, 8192, x1b, x2b),
    )
    for p, (rref, tsz, xab, xbb) in enumerate(planes):
        maskp = tsz - 1

        def lvl_body(l, _, rref=rref, tsz=tsz, xab=xab, xbb=xbb,
                     maskp=maskp, p=p):
            pltpu.sync_copy(rref.at[pl.ds(l * tsz, tsz)], rbuf.at[pl.ds(0, tsz)])
            scl = plsc.load_gather(sclb, [jnp.full((16,), l, jnp.int32)])

            def pt_body(g, _):
                n0 = g * 16
                pa = xab[pl.ds(n0, 16)] * scl + 0.5
                pb = xbb[pl.ds(n0, 16)] * scl + 0.5
                ia = pa.astype(jnp.int32)
                ib = pb.astype(jnp.int32)
                fa = pa - ia.astype(jnp.float32)
                fb = pb - ib.astype(jnp.float32)
                ga = 1.0 - fa
                gb2 = 1.0 - fb
                hb0 = ib * PRIME1
                hb1 = hb0 + PRIME1
                ia1 = ia + 1
                i00 = lax.bitwise_and(lax.bitwise_xor(ia, hb0), maskp)
                i10 = lax.bitwise_and(lax.bitwise_xor(ia1, hb0), maskp)
                i01 = lax.bitwise_and(lax.bitwise_xor(ia, hb1), maskp)
                i11 = lax.bitwise_and(lax.bitwise_xor(ia1, hb1), maskp)
                v = (ga * gb2) * plsc.load_gather(rbuf, [i00])
                v = v + (fa * gb2) * plsc.load_gather(rbuf, [i10])
                v = v + (ga * fb) * plsc.load_gather(rbuf, [i01])
                v = v + (fa * fb) * plsc.load_gather(rbuf, [i11])
                colb[pl.ds(n0, 16)] = v
                return _

            lax.fori_loop(0, PPT // 16, pt_body, None)
            pltpu.sync_copy(
                colb, d_out.at[pl.ds((p * 8 + l) * N_PTS + base, PPT)]
            )
            return _

        lax.fori_loop(0, N_LEVELS, lvl_body, None)


def _sc_call(xT, scal, wst2, R0, R1, R2):
    mesh = plsc.VectorSubcoreMesh(
        core_axis_name="c", subcore_axis_name="s", num_cores=2, num_subcores=16
    )
    fn = pl.kernel(
        _sc_body,
        out_type=(
            jax.ShapeDtypeStruct((N_LEVELS * 4 * N_PTS,), jnp.float32),
            jax.ShapeDtypeStruct((24 * N_PTS,), jnp.float32),
        ),
        mesh=mesh,
        compiler_params=pltpu.CompilerParams(
            needs_layout_passes=False, use_tc_tiling_on_sc=False
        ),
        scratch_types=[
            pltpu.VMEM((PPT,), jnp.float32),
            pltpu.VMEM((PPT,), jnp.float32),
            pltpu.VMEM((PPT,), jnp.float32),
            pltpu.VMEM((16,), jnp.float32),
            pltpu.VMEM((CH8,), jnp.int32),
            pltpu.VMEM((CH8 * 4,), jnp.int32),
            pltpu.VMEM((CH8,), jnp.float32),
            pltpu.VMEM((CH8 * 4,), jnp.float32),
            pltpu.VMEM((CH * 4,), jnp.float32),
            pltpu.VMEM((32768,), jnp.float32),
            pltpu.VMEM((PPT,), jnp.float32),
            pltpu.SemaphoreType.DMA,
        ],
    )
    return fn(xT, scal, wst2, R0, R1, R2)


def kernel(x, t, W_static, W_dyn0, W_dyn1, W_dyn2):
    ts = t[0]
    s = ts * (TIME_RES - 1.0)
    resid = (ts * float(TIME_RES) - s) - ts
    fl_s = jnp.floor(s)
    on_grid = s == fl_s
    i1 = fl_s.astype(jnp.int32) - jnp.where(on_grid & (resid < 0), 1, 0)
    i2 = jnp.ceil(s).astype(jnp.int32) + jnp.where(on_grid & (resid > 0), 1, 0)
    same = i1 == i2
    w1 = jnp.where(same, jnp.float32(1.0), i2.astype(jnp.float32) - s)
    w2 = s - i1.astype(jnp.float32)

    tn = [i / (NUM_BASIS - 1) for i in range(NUM_BASIS)]
    bs = []
    for j in range(NUM_BASIS):
        b = jnp.float32(1.0)
        for m in range(NUM_BASIS):
            if m != j:
                b = b * (ts - tn[m]) / (tn[j] - tn[m])
        bs.append(b)
    cvec = jnp.stack([w1 * bj for bj in bs] + [w2 * bj for bj in bs])
    cvec = cvec.astype(jnp.float32)

    def slice_t(w, i):
        return lax.dynamic_index_in_dim(w, i, axis=0, keepdims=False)

    rs = []
    for wdyn in (W_dyn0, W_dyn1, W_dyn2):
        tsz = wdyn.shape[2]
        wt = wdyn.transpose(0, 1, 3, 2)
        a3 = slice_t(wt, i1)
        b3 = slice_t(wt, i2)
        rs.append(_reduce_pair(cvec, a3, b3).reshape(N_LEVELS * tsz))

    xT = x.T.reshape(3 * N_PTS)
    scal = jnp.asarray(SCALES + [0.0] * 8, dtype=jnp.float32)
    wst2 = W_static.transpose(0, 2, 1).reshape(N_LEVELS * F * T_STATIC)

    s_flat, d_mat = _sc_call(xT, scal, wst2, rs[0], rs[1], rs[2])

    feat_static = (
        s_flat.reshape(N_LEVELS, N_PTS, F).transpose(1, 0, 2).reshape(N_PTS, 32)
    )
    feat_dynamic = d_mat.reshape(24, N_PTS).T
    return (feat_static, feat_dynamic)


# speedup vs baseline: 42.3822x; 1.2424x over previous
import functools

import numpy as np
import jax
import jax.numpy as jnp
from jax import lax
from jax.experimental import pallas as pl
from jax.experimental.pallas import tpu as pltpu
from jax.experimental.pallas import tpu_sc as plsc

N_LEVELS = 8
F = 4
BASE_RES = 512.0
MAX_RES = 32768.0
PER_LEVEL_SCALE = float(np.exp2(np.log2(MAX_RES / BASE_RES) / (N_LEVELS - 1)))
TIME_RES = 8
NUM_BASIS = 4
N_PTS = 131072

PRIME1 = int(np.uint32(2654435761).astype(np.int32))
PRIME2 = int(np.uint32(805459861).astype(np.int32))

SCALES = [
    float(np.exp2(l * np.log2(PER_LEVEL_SCALE)) * BASE_RES - 1.0)
    for l in range(N_LEVELS)
]

T_STATIC = 2 ** 19
MASK_S = T_STATIC - 1

NW = 32
PPT = N_PTS // NW
CH = 512
CH8 = CH * 8


def _reduce_body(c_ref, a_ref, b_ref, o_ref):
    a = a_ref[...]
    b = b_ref[...]
    acc = (
        c_ref[0] * a[:, 0, :] + c_ref[1] * a[:, 1, :]
        + c_ref[2] * a[:, 2, :] + c_ref[3] * a[:, 3, :]
        + c_ref[4] * b[:, 0, :] + c_ref[5] * b[:, 1, :]
        + c_ref[6] * b[:, 2, :] + c_ref[7] * b[:, 3, :]
    )
    o_ref[...] = acc


def _reduce_pair(cvec, a3, b3):
    t = a3.shape[2]
    blk = 4096
    return pl.pallas_call(
        _reduce_body,
        grid=(t // blk,),
        in_specs=[
            pl.BlockSpec(memory_space=pltpu.SMEM),
            pl.BlockSpec((N_LEVELS, 4, blk), lambda i: (0, 0, i)),
            pl.BlockSpec((N_LEVELS, 4, blk), lambda i: (0, 0, i)),
        ],
        out_specs=pl.BlockSpec((N_LEVELS, blk), lambda i: (0, i)),
        out_shape=jax.ShapeDtypeStruct((N_LEVELS, t), jnp.float32),
    )(cvec, a3, b3)


def _sc_body(xT, scal, wst, r0, r1, r2, s_out, d_out,
             x0b, x1b, x2b, sclb, ridxb, idxb, wb, gbuf, stg, rbuf, colb,
             semA, semB, semOA, semOB):
    wid = lax.axis_index("s") * 2 + lax.axis_index("c")
    base = wid * PPT

    pltpu.sync_copy(xT.at[pl.ds(base, PPT)], x0b)
    pltpu.sync_copy(xT.at[pl.ds(N_PTS + base, PPT)], x1b)
    pltpu.sync_copy(xT.at[pl.ds(2 * N_PTS + base, PPT)], x2b)
    pltpu.sync_copy(scal, sclb)

    iota = lax.iota(jnp.int32, 16)
    q4 = lax.shift_right_logical(iota, 2)
    f4 = lax.bitwise_and(iota, 3)

    EL = CH8 * 4
    CPL = PPT // CH
    NCH = N_LEVELS * CPL
    t4 = jnp.int32(4 * T_STATIC)
    f4t = f4 * T_STATIC

    def hfire(t, s, semS, scale=None):
        lvl = lax.shift_right_logical(t, 3)
        pbase = lax.bitwise_and(t, CPL - 1) * CH
        if scale is None:
            scale = plsc.load_gather(sclb, [jnp.full((16,), lvl, jnp.int32)])
        lvl_off = lvl * t4

        def hash_body(i, _):
            n0 = pbase + i * 16
            o = i * 16
            p0 = x0b[pl.ds(n0, 16)] * scale + 0.5
            p1 = x1b[pl.ds(n0, 16)] * scale + 0.5
            p2 = x2b[pl.ds(n0, 16)] * scale + 0.5
            i0 = p0.astype(jnp.int32)
            i1_ = p1.astype(jnp.int32)
            i2_ = p2.astype(jnp.int32)
            f0 = p0 - i0.astype(jnp.float32)
            f1 = p1 - i1_.astype(jnp.float32)
            f2 = p2 - i2_.astype(jnp.float32)
            g0 = 1.0 - f0
            g1 = 1.0 - f1
            g2 = 1.0 - f2
            h1a = i1_ * PRIME1
            h1b = h1a + PRIME1
            h2a = i2_ * PRIME2
            h2b = h2a + PRIME2
            i0b = i0 + 1
            for c in range(8):
                hx = i0b if (c & 1) else i0
                hy = h1b if (c & 2) else h1a
                hz = h2b if (c & 4) else h2a
                idx = lax.bitwise_and(
                    lax.bitwise_xor(lax.bitwise_xor(hx, hy), hz), MASK_S
                ) + lvl_off
                wx = f0 if (c & 1) else g0
                wy = f1 if (c & 2) else g1
                wz = f2 if (c & 4) else g2
                ridxb[pl.ds(c * CH + o, 16)] = idx
                wb[pl.ds(s * CH8 + c * CH + o, 16)] = wx * wy * wz
            return _

        lax.fori_loop(0, CH // 16, hash_body, None)

        def expand_body(g, _):
            for q in range(4):
                piece = plsc.load_gather(ridxb, [g * 16 + q * 4 + q4])
                idxb[pl.ds(s * EL + g * 64 + q * 16, 16)] = piece + f4t
            return _

        lax.fori_loop(0, CH8 // 16, expand_body, None)
        pltpu.async_copy(
            wst.at[idxb.at[pl.ds(s * EL, EL)]],
            gbuf.at[pl.ds(s * EL, EL)], semS,
        )

    def accum(t, s, u, semS, semO):
        pltpu.make_async_copy(
            wst.at[idxb.at[pl.ds(s * EL, EL)]],
            gbuf.at[pl.ds(s * EL, EL)], semS,
        ).wait()
        lvl = lax.shift_right_logical(t, 3)
        pbase = lax.bitwise_and(t, CPL - 1) * CH
        dst = s_out.at[pl.ds(lvl * (4 * N_PTS) + (base + pbase) * 4, CH * 4)]

        @pl.when(u >= 1)
        def _():
            pltpu.make_async_copy(stg.at[pl.ds(s * CH * 4, CH * 4)], dst,
                                  semO).wait()

        def acc_body(g, _):
            gb16 = g * 16
            for q in range(4):
                off = gb16 + q * 4
                acc = jnp.zeros((16,), jnp.float32)
                for c in range(8):
                    row = gbuf[pl.ds(s * EL + (c * CH + off) * 4, 16)]
                    wexp = plsc.load_gather(wb, [s * CH8 + c * CH + off + q4])
                    acc = acc + wexp * row
                stg[pl.ds(s * CH * 4 + gb16 * 4 + q * 16, 16)] = acc
            return _

        lax.fori_loop(0, CH // 16, acc_body, None)
        pltpu.async_copy(stg.at[pl.ds(s * CH * 4, CH * 4)], dst, semO)

    hfire(jnp.int32(0), 0, semA,
          scale=jnp.full((16,), SCALES[0], jnp.float32))

    def body2(u, _):
        t0 = u * 2
        hfire(t0 + 1, 1, semB)
        accum(t0, 0, u, semA, semOA)

        @pl.when(t0 + 2 < NCH)
        def _():
            hfire(t0 + 2, 0, semA)

        accum(t0 + 1, 1, u, semB, semOB)
        return _

    lax.fori_loop(0, NCH // 2, body2, None)
    dummy = s_out.at[pl.ds(0, CH * 4)]
    pltpu.make_async_copy(stg.at[pl.ds(0, CH * 4)], dummy, semOA).wait()
    pltpu.make_async_copy(stg.at[pl.ds(CH * 4, CH * 4)], dummy, semOB).wait()

    planes = (
        (r0, 32768, x0b, x1b),
        (r1, 8192, x0b, x2b),
        (r2, 8192, x1b, x2b),
    )
    for p, (rref, tsz, xab, xbb) in enumerate(planes):
        maskp = tsz - 1

        def lvl_body(l, _, rref=rref, tsz=tsz, xab=xab, xbb=xbb,
                     maskp=maskp, p=p):
            pltpu.sync_copy(rref.at[pl.ds(l * tsz, tsz)], rbuf.at[pl.ds(0, tsz)])
            scl = plsc.load_gather(sclb, [jnp.full((16,), l, jnp.int32)])

            def pt_body(g, _, hh):
                n0 = hh * (PPT // 2) + g * 16
                pa = xab[pl.ds(n0, 16)] * scl + 0.5
                pb = xbb[pl.ds(n0, 16)] * scl + 0.5
                ia = pa.astype(jnp.int32)
                ib = pb.astype(jnp.int32)
                fa = pa - ia.astype(jnp.float32)
                fb = pb - ib.astype(jnp.float32)
                ga = 1.0 - fa
                gb2 = 1.0 - fb
                hb0 = ib * PRIME1
                hb1 = hb0 + PRIME1
                ia1 = ia + 1
                i00 = lax.bitwise_and(lax.bitwise_xor(ia, hb0), maskp)
                i10 = lax.bitwise_and(lax.bitwise_xor(ia1, hb0), maskp)
                i01 = lax.bitwise_and(lax.bitwise_xor(ia, hb1), maskp)
                i11 = lax.bitwise_and(lax.bitwise_xor(ia1, hb1), maskp)
                v = (ga * gb2) * plsc.load_gather(rbuf, [i00])
                v = v + (fa * gb2) * plsc.load_gather(rbuf, [i10])
                v = v + (ga * fb) * plsc.load_gather(rbuf, [i01])
                v = v + (fa * fb) * plsc.load_gather(rbuf, [i11])
                colb[pl.ds(g * 16, 16)] = v
                return _

            import functools as _ft
            for hh in range(2):
                lax.fori_loop(0, PPT // 32, _ft.partial(pt_body, hh=hh), None)
                pltpu.sync_copy(
                    colb,
                    d_out.at[pl.ds((p * 8 + l) * N_PTS + base
                                   + hh * (PPT // 2), PPT // 2)],
                )
            return _

        lax.fori_loop(0, N_LEVELS, lvl_body, None)


def _sc_call(xT, scal, wst2, R0, R1, R2):
    mesh = plsc.VectorSubcoreMesh(
        core_axis_name="c", subcore_axis_name="s", num_cores=2, num_subcores=16
    )
    fn = pl.kernel(
        _sc_body,
        out_type=(
            jax.ShapeDtypeStruct((N_LEVELS * 4 * N_PTS,), jnp.float32),
            jax.ShapeDtypeStruct((24 * N_PTS,), jnp.float32),
        ),
        mesh=mesh,
        compiler_params=pltpu.CompilerParams(
            needs_layout_passes=False, use_tc_tiling_on_sc=False
        ),
        scratch_types=[
            pltpu.VMEM((PPT,), jnp.float32),
            pltpu.VMEM((PPT,), jnp.float32),
            pltpu.VMEM((PPT,), jnp.float32),
            pltpu.VMEM((16,), jnp.float32),
            pltpu.VMEM((CH8,), jnp.int32),
            pltpu.VMEM((2 * CH8 * 4,), jnp.int32),
            pltpu.VMEM((2 * CH8,), jnp.float32),
            pltpu.VMEM((2 * CH8 * 4,), jnp.float32),
            pltpu.VMEM((2 * CH * 4,), jnp.float32),
            pltpu.VMEM((32768,), jnp.float32),
            pltpu.VMEM((PPT // 2,), jnp.float32),
            pltpu.SemaphoreType.DMA,
            pltpu.SemaphoreType.DMA,
            pltpu.SemaphoreType.DMA,
            pltpu.SemaphoreType.DMA,
        ],
    )
    return fn(xT, scal, wst2, R0, R1, R2)


def kernel(x, t, W_static, W_dyn0, W_dyn1, W_dyn2):
    ts = t[0]
    s = ts * (TIME_RES - 1.0)
    resid = (ts * float(TIME_RES) - s) - ts
    fl_s = jnp.floor(s)
    on_grid = s == fl_s
    i1 = fl_s.astype(jnp.int32) - jnp.where(on_grid & (resid < 0), 1, 0)
    i2 = jnp.ceil(s).astype(jnp.int32) + jnp.where(on_grid & (resid > 0), 1, 0)
    same = i1 == i2
    w1 = jnp.where(same, jnp.float32(1.0), i2.astype(jnp.float32) - s)
    w2 = s - i1.astype(jnp.float32)

    tn = [i / (NUM_BASIS - 1) for i in range(NUM_BASIS)]
    bs = []
    for j in range(NUM_BASIS):
        b = jnp.float32(1.0)
        for m in range(NUM_BASIS):
            if m != j:
                b = b * (ts - tn[m]) / (tn[j] - tn[m])
        bs.append(b)
    cvec = jnp.stack([w1 * bj for bj in bs] + [w2 * bj for bj in bs])
    cvec = cvec.astype(jnp.float32)

    def slice_t(w, i):
        return lax.dynamic_index_in_dim(w, i, axis=0, keepdims=False)

    rs = []
    for wdyn in (W_dyn0, W_dyn1, W_dyn2):
        tsz = wdyn.shape[2]
        wt = wdyn.transpose(0, 1, 3, 2)
        a3 = slice_t(wt, i1)
        b3 = slice_t(wt, i2)
        rs.append(_reduce_pair(cvec, a3, b3).reshape(N_LEVELS * tsz))

    xT = x.T.reshape(3 * N_PTS)
    scal = jnp.asarray(SCALES + [0.0] * 8, dtype=jnp.float32)
    wst2 = W_static.transpose(0, 2, 1).reshape(N_LEVELS * F * T_STATIC)

    s_flat, d_mat = _sc_call(xT, scal, wst2, rs[0], rs[1], rs[2])

    feat_static = (
        s_flat.reshape(N_LEVELS, N_PTS, F).transpose(1, 0, 2).reshape(N_PTS, 32)
    )
    feat_dynamic = d_mat.reshape(24, N_PTS).T
    return (feat_static, feat_dynamic)


# speedup vs baseline: 44.6563x; 1.0537x over previous
import functools

import numpy as np
import jax
import jax.numpy as jnp
from jax import lax
from jax.experimental import pallas as pl
from jax.experimental.pallas import tpu as pltpu
from jax.experimental.pallas import tpu_sc as plsc

N_LEVELS = 8
F = 4
BASE_RES = 512.0
MAX_RES = 32768.0
PER_LEVEL_SCALE = float(np.exp2(np.log2(MAX_RES / BASE_RES) / (N_LEVELS - 1)))
TIME_RES = 8
NUM_BASIS = 4
N_PTS = 131072

PRIME1 = int(np.uint32(2654435761).astype(np.int32))
PRIME2 = int(np.uint32(805459861).astype(np.int32))

SCALES = [
    float(np.exp2(l * np.log2(PER_LEVEL_SCALE)) * BASE_RES - 1.0)
    for l in range(N_LEVELS)
]

T_STATIC = 2 ** 19
MASK_S = T_STATIC - 1

NW = 32
PPT = N_PTS // NW
CH = 512
CH8 = CH * 8


def _reduce_body(c_ref, a_ref, b_ref, o_ref):
    a = a_ref[...]
    b = b_ref[...]
    acc = (
        c_ref[0] * a[:, 0, :] + c_ref[1] * a[:, 1, :]
        + c_ref[2] * a[:, 2, :] + c_ref[3] * a[:, 3, :]
        + c_ref[4] * b[:, 0, :] + c_ref[5] * b[:, 1, :]
        + c_ref[6] * b[:, 2, :] + c_ref[7] * b[:, 3, :]
    )
    o_ref[...] = acc


def _reduce_pair(cvec, a3, b3):
    t = a3.shape[2]
    blk = 4096
    return pl.pallas_call(
        _reduce_body,
        grid=(t // blk,),
        in_specs=[
            pl.BlockSpec(memory_space=pltpu.SMEM),
            pl.BlockSpec((N_LEVELS, 4, blk), lambda i: (0, 0, i)),
            pl.BlockSpec((N_LEVELS, 4, blk), lambda i: (0, 0, i)),
        ],
        out_specs=pl.BlockSpec((N_LEVELS, blk), lambda i: (0, i)),
        out_shape=jax.ShapeDtypeStruct((N_LEVELS, t), jnp.float32),
    )(cvec, a3, b3)


def _sc_static_body(xT, scal, wst, s_out,
                    x0b, x1b, x2b, sclb, ridxb, idxb, wb, gbuf, stg,
                    semA, semB, semOA, semOB):
    wid = lax.axis_index("s") * 2 + lax.axis_index("c")
    base = wid * PPT

    pltpu.sync_copy(xT.at[pl.ds(base, PPT)], x0b)
    pltpu.sync_copy(xT.at[pl.ds(N_PTS + base, PPT)], x1b)
    pltpu.sync_copy(xT.at[pl.ds(2 * N_PTS + base, PPT)], x2b)
    pltpu.sync_copy(scal, sclb)

    iota = lax.iota(jnp.int32, 16)
    q4 = lax.shift_right_logical(iota, 2)
    f4 = lax.bitwise_and(iota, 3)

    EL = CH8 * 4
    CPL = PPT // CH
    NCH = N_LEVELS * CPL
    t4 = jnp.int32(4 * T_STATIC)
    f4t = f4 * T_STATIC

    def hfire(t, s, semS, scale=None):
        lvl = lax.shift_right_logical(t, 3)
        pbase = lax.bitwise_and(t, CPL - 1) * CH
        if scale is None:
            scale = plsc.load_gather(sclb, [jnp.full((16,), lvl, jnp.int32)])
        lvl_off = lvl * t4

        def hash_body(i, _):
            n0 = pbase + i * 16
            o = i * 16
            p0 = x0b[pl.ds(n0, 16)] * scale + 0.5
            p1 = x1b[pl.ds(n0, 16)] * scale + 0.5
            p2 = x2b[pl.ds(n0, 16)] * scale + 0.5
            i0 = p0.astype(jnp.int32)
            i1_ = p1.astype(jnp.int32)
            i2_ = p2.astype(jnp.int32)
            f0 = p0 - i0.astype(jnp.float32)
            f1 = p1 - i1_.astype(jnp.float32)
            f2 = p2 - i2_.astype(jnp.float32)
            g0 = 1.0 - f0
            g1 = 1.0 - f1
            g2 = 1.0 - f2
            h1a = i1_ * PRIME1
            h1b = h1a + PRIME1
            h2a = i2_ * PRIME2
            h2b = h2a + PRIME2
            i0b = i0 + 1
            for c in range(8):
                hx = i0b if (c & 1) else i0
                hy = h1b if (c & 2) else h1a
                hz = h2b if (c & 4) else h2a
                idx = lax.bitwise_and(
                    lax.bitwise_xor(lax.bitwise_xor(hx, hy), hz), MASK_S
                ) + lvl_off
                wx = f0 if (c & 1) else g0
                wy = f1 if (c & 2) else g1
                wz = f2 if (c & 4) else g2
                ridxb[pl.ds(c * CH + o, 16)] = idx
                wb[pl.ds(s * CH8 + c * CH + o, 16)] = wx * wy * wz
            return _

        lax.fori_loop(0, CH // 16, hash_body, None)

        def expand_body(g, _):
            for q in range(4):
                piece = plsc.load_gather(ridxb, [g * 16 + q * 4 + q4])
                idxb[pl.ds(s * EL + g * 64 + q * 16, 16)] = piece + f4t
            return _

        lax.fori_loop(0, CH8 // 16, expand_body, None)
        pltpu.async_copy(
            wst.at[idxb.at[pl.ds(s * EL, EL)]],
            gbuf.at[pl.ds(s * EL, EL)], semS,
        )

    def accum(t, s, u, semS, semO):
        pltpu.make_async_copy(
            wst.at[idxb.at[pl.ds(s * EL, EL)]],
            gbuf.at[pl.ds(s * EL, EL)], semS,
        ).wait()
        lvl = lax.shift_right_logical(t, 3)
        pbase = lax.bitwise_and(t, CPL - 1) * CH
        dst = s_out.at[pl.ds(lvl * (4 * N_PTS) + (base + pbase) * 4, CH * 4)]

        @pl.when(u >= 1)
        def _():
            pltpu.make_async_copy(stg.at[pl.ds(s * CH * 4, CH * 4)], dst,
                                  semO).wait()

        def acc_body(g, _):
            gb16 = g * 16
            for q in range(4):
                off = gb16 + q * 4
                acc = jnp.zeros((16,), jnp.float32)
                for c in range(8):
                    row = gbuf[pl.ds(s * EL + (c * CH + off) * 4, 16)]
                    wexp = plsc.load_gather(wb, [s * CH8 + c * CH + off + q4])
                    acc = acc + wexp * row
                stg[pl.ds(s * CH * 4 + gb16 * 4 + q * 16, 16)] = acc
            return _

        lax.fori_loop(0, CH // 16, acc_body, None)
        pltpu.async_copy(stg.at[pl.ds(s * CH * 4, CH * 4)], dst, semO)

    hfire(jnp.int32(0), 0, semA,
          scale=jnp.full((16,), SCALES[0], jnp.float32))

    def body2(u, _):
        t0 = u * 2
        hfire(t0 + 1, 1, semB)
        accum(t0, 0, u, semA, semOA)

        @pl.when(t0 + 2 < NCH)
        def _():
            hfire(t0 + 2, 0, semA)

        accum(t0 + 1, 1, u, semB, semOB)
        return _

    lax.fori_loop(0, NCH // 2, body2, None)
    dummy = s_out.at[pl.ds(0, CH * 4)]
    pltpu.make_async_copy(stg.at[pl.ds(0, CH * 4)], dummy, semOA).wait()
    pltpu.make_async_copy(stg.at[pl.ds(CH * 4, CH * 4)], dummy, semOB).wait()


def _sc_dyn_body(xT, scal, r0, r1, r2, d_out,
                 x0b, x1b, x2b, sclb, rbuf, colb):
    wid = lax.axis_index("s") * 2 + lax.axis_index("c")
    base = wid * PPT

    pltpu.sync_copy(xT.at[pl.ds(base, PPT)], x0b)
    pltpu.sync_copy(xT.at[pl.ds(N_PTS + base, PPT)], x1b)
    pltpu.sync_copy(xT.at[pl.ds(2 * N_PTS + base, PPT)], x2b)
    pltpu.sync_copy(scal, sclb)

    planes = (
        (r0, 32768, x0b, x1b),
        (r1, 8192, x0b, x2b),
        (r2, 8192, x1b, x2b),
    )
    for p, (rref, tsz, xab, xbb) in enumerate(planes):
        maskp = tsz - 1

        def lvl_body(l, _, rref=rref, tsz=tsz, xab=xab, xbb=xbb,
                     maskp=maskp, p=p):
            pltpu.sync_copy(rref.at[pl.ds(l * tsz, tsz)], rbuf.at[pl.ds(0, tsz)])
            scl = plsc.load_gather(sclb, [jnp.full((16,), l, jnp.int32)])

            def pt_body(g, _, hh):
                n0 = hh * (PPT // 2) + g * 16
                pa = xab[pl.ds(n0, 16)] * scl + 0.5
                pb = xbb[pl.ds(n0, 16)] * scl + 0.5
                ia = pa.astype(jnp.int32)
                ib = pb.astype(jnp.int32)
                fa = pa - ia.astype(jnp.float32)
                fb = pb - ib.astype(jnp.float32)
                ga = 1.0 - fa
                gb2 = 1.0 - fb
                hb0 = ib * PRIME1
                hb1 = hb0 + PRIME1
                ia1 = ia + 1
                i00 = lax.bitwise_and(lax.bitwise_xor(ia, hb0), maskp)
                i10 = lax.bitwise_and(lax.bitwise_xor(ia1, hb0), maskp)
                i01 = lax.bitwise_and(lax.bitwise_xor(ia, hb1), maskp)
                i11 = lax.bitwise_and(lax.bitwise_xor(ia1, hb1), maskp)
                v = (ga * gb2) * plsc.load_gather(rbuf, [i00])
                v = v + (fa * gb2) * plsc.load_gather(rbuf, [i10])
                v = v + (ga * fb) * plsc.load_gather(rbuf, [i01])
                v = v + (fa * fb) * plsc.load_gather(rbuf, [i11])
                colb[pl.ds(g * 16, 16)] = v
                return _

            import functools as _ft
            for hh in range(2):
                lax.fori_loop(0, PPT // 32, _ft.partial(pt_body, hh=hh), None)
                pltpu.sync_copy(
                    colb,
                    d_out.at[pl.ds((p * 8 + l) * N_PTS + base
                                   + hh * (PPT // 2), PPT // 2)],
                )
            return _

        lax.fori_loop(0, N_LEVELS, lvl_body, None)


def _sc_call(xT, scal, wst2, R0, R1, R2):
    mesh = plsc.VectorSubcoreMesh(
        core_axis_name="c", subcore_axis_name="s", num_cores=2, num_subcores=16
    )
    cp = pltpu.CompilerParams(
        needs_layout_passes=False, use_tc_tiling_on_sc=False
    )
    fn_s = pl.kernel(
        _sc_static_body,
        out_type=jax.ShapeDtypeStruct((N_LEVELS * 4 * N_PTS,), jnp.float32),
        mesh=mesh,
        compiler_params=cp,
        scratch_types=[
            pltpu.VMEM((PPT,), jnp.float32),
            pltpu.VMEM((PPT,), jnp.float32),
            pltpu.VMEM((PPT,), jnp.float32),
            pltpu.VMEM((16,), jnp.float32),
            pltpu.VMEM((CH8,), jnp.int32),
            pltpu.VMEM((2 * CH8 * 4,), jnp.int32),
            pltpu.VMEM((2 * CH8,), jnp.float32),
            pltpu.VMEM((2 * CH8 * 4,), jnp.float32),
            pltpu.VMEM((2 * CH * 4,), jnp.float32),
            pltpu.SemaphoreType.DMA,
            pltpu.SemaphoreType.DMA,
            pltpu.SemaphoreType.DMA,
            pltpu.SemaphoreType.DMA,
        ],
    )
    fn_d = pl.kernel(
        _sc_dyn_body,
        out_type=jax.ShapeDtypeStruct((24 * N_PTS,), jnp.float32),
        mesh=mesh,
        compiler_params=cp,
        scratch_types=[
            pltpu.VMEM((PPT,), jnp.float32),
            pltpu.VMEM((PPT,), jnp.float32),
            pltpu.VMEM((PPT,), jnp.float32),
            pltpu.VMEM((16,), jnp.float32),
            pltpu.VMEM((32768,), jnp.float32),
            pltpu.VMEM((PPT // 2,), jnp.float32),
        ],
    )
    return fn_s(xT, scal, wst2), fn_d(xT, scal, R0, R1, R2)


def kernel(x, t, W_static, W_dyn0, W_dyn1, W_dyn2):
    ts = t[0]
    s = ts * (TIME_RES - 1.0)
    resid = (ts * float(TIME_RES) - s) - ts
    fl_s = jnp.floor(s)
    on_grid = s == fl_s
    i1 = fl_s.astype(jnp.int32) - jnp.where(on_grid & (resid < 0), 1, 0)
    i2 = jnp.ceil(s).astype(jnp.int32) + jnp.where(on_grid & (resid > 0), 1, 0)
    same = i1 == i2
    w1 = jnp.where(same, jnp.float32(1.0), i2.astype(jnp.float32) - s)
    w2 = s - i1.astype(jnp.float32)

    tn = [i / (NUM_BASIS - 1) for i in range(NUM_BASIS)]
    bs = []
    for j in range(NUM_BASIS):
        b = jnp.float32(1.0)
        for m in range(NUM_BASIS):
            if m != j:
                b = b * (ts - tn[m]) / (tn[j] - tn[m])
        bs.append(b)
    cvec = jnp.stack([w1 * bj for bj in bs] + [w2 * bj for bj in bs])
    cvec = cvec.astype(jnp.float32)

    def slice_t(w, i):
        return lax.dynamic_index_in_dim(w, i, axis=0, keepdims=False)

    rs = []
    for wdyn in (W_dyn0, W_dyn1, W_dyn2):
        tsz = wdyn.shape[2]
        wt = wdyn.transpose(0, 1, 3, 2)
        a3 = slice_t(wt, i1)
        b3 = slice_t(wt, i2)
        rs.append(_reduce_pair(cvec, a3, b3).reshape(N_LEVELS * tsz))

    xT = x.T.reshape(3 * N_PTS)
    scal = jnp.asarray(SCALES + [0.0] * 8, dtype=jnp.float32)
    wst2 = W_static.transpose(0, 2, 1).reshape(N_LEVELS * F * T_STATIC)

    s_flat, d_mat = _sc_call(xT, scal, wst2, rs[0], rs[1], rs[2])

    feat_static = (
        s_flat.reshape(N_LEVELS, N_PTS, F).transpose(1, 0, 2).reshape(N_PTS, 32)
    )
    feat_dynamic = d_mat.reshape(24, N_PTS).T
    return (feat_static, feat_dynamic)
